# baseline (speedup 1.0000x reference)
import functools

import jax
import jax.numpy as jnp
from jax.experimental import pallas as pl
from jax.experimental.pallas import tpu as pltpu

_INV_TS = 10.0
_INV_TT = 25.0

_R = 8
_SUB = 64
_LANE = 128


def _loss_body(mask_ref, s_ref, t_ref, c_ref, out_ref, acc_ref, nacc_ref):
    i = pl.program_id(0)

    @pl.when(i == 0)
    def _init():
        acc_ref[0] = 0.0
        nacc_ref[0] = 0.0

    s = s_ref[...]
    t = t_ref[...]
    c = c_ref[...]
    m = mask_ref[0, 0, :]

    z = (t - c) * _INV_TT
    zmax = jnp.max(z, axis=(1, 2), keepdims=True)
    e = jnp.exp(z - zmax)
    esum = jnp.sum(e, axis=(1, 2))

    y = s * _INV_TS
    ymax = jnp.max(y, axis=(1, 2), keepdims=True)
    ysum = jnp.sum(jnp.exp(y - ymax), axis=(1, 2))

    dot = jnp.sum(e * y, axis=(1, 2))
    per_token = -(dot / esum) + ymax[:, 0, 0] + jnp.log(ysum)

    acc_ref[0] += jnp.sum(per_token * m)
    nacc_ref[0] += jnp.sum(m)

    @pl.when(i == pl.num_programs(0) - 1)
    def _fin():
        out_ref[0] = acc_ref[0] / jnp.maximum(nacc_ref[0], 1.0)


def kernel(student_patch_out, teacher_patch_out, mask, center):
    B, N, D = student_patch_out.shape
    BN = B * N
    n_steps = BN // _R
    s3 = student_patch_out.reshape(BN, _SUB, _LANE)
    t3 = teacher_patch_out.reshape(BN, _SUB, _LANE)
    c3 = center.reshape(1, _SUB, _LANE)
    m3 = mask.reshape(n_steps, 1, _R).astype(jnp.float32)

    out = pl.pallas_call(
        _loss_body,
        grid=(n_steps,),
        in_specs=[
            pl.BlockSpec((1, 1, _R), lambda i: (i, 0, 0)),
            pl.BlockSpec((_R, _SUB, _LANE), lambda i: (i, 0, 0)),
            pl.BlockSpec((_R, _SUB, _LANE), lambda i: (i, 0, 0)),
            pl.BlockSpec((1, _SUB, _LANE), lambda i: (0, 0, 0)),
        ],
        out_specs=pl.BlockSpec(memory_space=pltpu.SMEM),
        out_shape=jax.ShapeDtypeStruct((1,), jnp.float32),
        scratch_shapes=[
            pltpu.SMEM((1,), jnp.float32),
            pltpu.SMEM((1,), jnp.float32),
        ],
    )(m3, s3, t3, c3)
    return out[0]
